# baseline (speedup 1.0000x reference)
import functools

import jax
import jax.numpy as jnp
from jax import lax
from jax.experimental import pallas as pl
from jax.experimental.pallas import tpu as pltpu
from jax.experimental.pallas import tpu_sc as plsc

B, C, H, W = 16, 1024, 32, 32
P = H * W
D = 256
L = 16
NC, NS = 2, 16
HALF = P // 2
PP = 16
NPC = HALF // PP
NV = C // L
K = 128
NK = HALF // K


def _sc_body(probs_hbm, emb_hbm, x_hbm, tok_hbm,
             pbuf0, pbuf1, rows0, rows1, bi,
             psem0, psem1, gsem0, gsem1, wsem0, wsem1):
    c = lax.axis_index("c")
    s = lax.axis_index("s")
    b = c * (B // NC) + s // 2
    p0 = (s % 2) * HALF

    iota = lax.iota(jnp.int32, L)

    def start_chunk(buf, sem, chunk_id):
        off = pl.multiple_of(p0 + chunk_id * PP, PP)
        pltpu.make_async_copy(
            probs_hbm.at[b, pl.ds(off, PP), :], buf, sem).start()

    def wait_chunk(buf, sem):
        pltpu.make_async_copy(
            probs_hbm.at[0, pl.ds(0, PP), :], buf, sem).wait()

    def process(buf, pbase):
        def pos_body(j):
            curv = buf[j, pl.ds(0, L)]
            curi = jnp.zeros((L,), jnp.int32)
            for v in range(1, NV):
                a = buf[j, pl.ds(v * L, L)]
                m = jnp.maximum(a, curv)
                upd = m != curv
                curi = jnp.where(upd, jnp.full((L,), v, jnp.int32), curi)
                curv = m
            gm = jnp.max(curv)
            chan = curi * L + iota
            cand = jnp.where(curv == jnp.full((L,), 1.0, jnp.float32) * gm,
                             chan, jnp.full((L,), C, jnp.int32))
            mn = jnp.full((L,), 1, jnp.int32) * jnp.min(cand)
            dst = jnp.full((L,), 1, jnp.int32) * (pbase + j)
            plsc.store_scatter(bi, [dst], mn, mask=iota == 0)

        plsc.parallel_loop(0, PP, unroll=8)(pos_body)

    start_chunk(pbuf0, psem0, 0)
    start_chunk(pbuf1, psem1, 1)

    def ring(g, _):
        wait_chunk(pbuf0, psem0)
        process(pbuf0, 2 * g * PP)

        @pl.when(2 * g + 2 < NPC)
        def _():
            start_chunk(pbuf0, psem0, 2 * g + 2)

        wait_chunk(pbuf1, psem1)
        process(pbuf1, (2 * g + 1) * PP)

        @pl.when(2 * g + 3 < NPC)
        def _():
            start_chunk(pbuf1, psem1, 2 * g + 3)

        return 0

    lax.fori_loop(0, NPC // 2, ring, 0)

    pltpu.sync_copy(bi, tok_hbm.at[b, pl.ds(p0, HALF)])

    rows = (rows0, rows1)
    gsems = (gsem0, gsem1)
    wsems = (wsem0, wsem1)
    wcopies = [None, None]
    for k in range(NK):
        par = k % 2
        if wcopies[par] is not None:
            wcopies[par].wait()
        pltpu.async_copy(
            emb_hbm.at[bi.at[pl.ds(k * K, K)]], rows[par], gsems[par]
        ).wait()
        wcopies[par] = pltpu.async_copy(
            rows[par], x_hbm.at[b, pl.ds(p0 + k * K, K), :], wsems[par])
    for wc in wcopies:
        if wc is not None:
            wc.wait()


def kernel(probs, tokens_shift, embedding_weight):
    probs_t = jnp.transpose(probs, (0, 2, 3, 1)).reshape(B, P, C)
    mesh = plsc.VectorSubcoreMesh(core_axis_name="c", subcore_axis_name="s")
    sc_call = functools.partial(
        pl.kernel, _sc_body, mesh=mesh,
        out_type=[
            jax.ShapeDtypeStruct((B, P, D), jnp.float32),
            jax.ShapeDtypeStruct((B, P), jnp.int32),
        ],
        scratch_types=[
            pltpu.VMEM((PP, C), jnp.float32),
            pltpu.VMEM((PP, C), jnp.float32),
            pltpu.VMEM((K, D), jnp.float32),
            pltpu.VMEM((K, D), jnp.float32),
            pltpu.VMEM((HALF,), jnp.int32),
            pltpu.SemaphoreType.DMA,
            pltpu.SemaphoreType.DMA,
            pltpu.SemaphoreType.DMA,
            pltpu.SemaphoreType.DMA,
            pltpu.SemaphoreType.DMA,
            pltpu.SemaphoreType.DMA,
        ],
        compiler_params=pltpu.CompilerParams(needs_layout_passes=False),
    )()
    x_rows, tok_raw = sc_call(probs_t, embedding_weight)
    tok = tok_raw + jnp.asarray(tokens_shift, jnp.int32)
    x = jnp.transpose(x_rows.reshape(B, H, W, D), (0, 3, 1, 2))
    return (x, tok.reshape(B, H, W))


# speedup vs baseline: 1.7779x; 1.7779x over previous
import functools

import jax
import jax.numpy as jnp
from jax import lax
from jax.experimental import pallas as pl
from jax.experimental.pallas import tpu as pltpu
from jax.experimental.pallas import tpu_sc as plsc

B, C, H, W = 16, 1024, 32, 32
P = H * W
D = 256
L = 16
NC, NS = 2, 16
HALF = P // 2
PP = 16
NPC = HALF // PP
NV = C // L
K = 128
NK = HALF // K


def _sc_body(probs_hbm, emb_hbm, x_hbm, tok_hbm,
             pbuf0, pbuf1, rows0, rows1, bi,
             psem0, psem1, gsem0, gsem1, wsem0, wsem1):
    c = lax.axis_index("c")
    s = lax.axis_index("s")
    b = c * (B // NC) + s // 2
    p0 = (s % 2) * HALF

    iota = lax.iota(jnp.int32, L)

    def start_chunk(buf, sem, chunk_id):
        off = pl.multiple_of(p0 + chunk_id * PP, PP)
        pltpu.make_async_copy(
            probs_hbm.at[b, pl.ds(off, PP), :], buf, sem).start()

    def wait_chunk(buf, sem):
        pltpu.make_async_copy(
            probs_hbm.at[0, pl.ds(0, PP), :], buf, sem).wait()

    def process(buf, pbase):
        def pos_body(j):
            curv = buf[j, pl.ds(0, L)]
            curi = jnp.zeros((L,), jnp.int32)
            for v in range(1, NV):
                a = buf[j, pl.ds(v * L, L)]
                m = jnp.maximum(a, curv)
                upd = m != curv
                curi = jnp.where(upd, jnp.full((L,), v, jnp.int32), curi)
                curv = m
            gm = jnp.max(curv)
            chan = curi * L + iota
            cand = jnp.where(curv == jnp.full((L,), 1.0, jnp.float32) * gm,
                             chan, jnp.full((L,), C, jnp.int32))
            mn = jnp.full((L,), 1, jnp.int32) * jnp.min(cand)
            dst = jnp.full((L,), 1, jnp.int32) * (pbase + j)
            plsc.store_scatter(bi, [dst], mn, mask=iota == 0)

        plsc.parallel_loop(0, PP, unroll=2)(pos_body)

    start_chunk(pbuf0, psem0, 0)
    start_chunk(pbuf1, psem1, 1)

    def ring(g, _):
        wait_chunk(pbuf0, psem0)
        process(pbuf0, 2 * g * PP)

        @pl.when(2 * g + 2 < NPC)
        def _():
            start_chunk(pbuf0, psem0, 2 * g + 2)

        wait_chunk(pbuf1, psem1)
        process(pbuf1, (2 * g + 1) * PP)

        @pl.when(2 * g + 3 < NPC)
        def _():
            start_chunk(pbuf1, psem1, 2 * g + 3)

        return 0

    lax.fori_loop(0, NPC // 2, ring, 0)

    pltpu.sync_copy(bi, tok_hbm.at[b, pl.ds(p0, HALF)])

    rows = (rows0, rows1)
    gsems = (gsem0, gsem1)
    wsems = (wsem0, wsem1)
    wcopies = [None, None]
    for k in range(NK):
        par = k % 2
        if wcopies[par] is not None:
            wcopies[par].wait()
        pltpu.async_copy(
            emb_hbm.at[bi.at[pl.ds(k * K, K)]], rows[par], gsems[par]
        ).wait()
        wcopies[par] = pltpu.async_copy(
            rows[par], x_hbm.at[b, pl.ds(p0 + k * K, K), :], wsems[par])
    for wc in wcopies:
        if wc is not None:
            wc.wait()


def kernel(probs, tokens_shift, embedding_weight):
    probs_t = jnp.transpose(probs, (0, 2, 3, 1)).reshape(B, P, C)
    mesh = plsc.VectorSubcoreMesh(core_axis_name="c", subcore_axis_name="s")
    sc_call = functools.partial(
        pl.kernel, _sc_body, mesh=mesh,
        out_type=[
            jax.ShapeDtypeStruct((B, P, D), jnp.float32),
            jax.ShapeDtypeStruct((B, P), jnp.int32),
        ],
        scratch_types=[
            pltpu.VMEM((PP, C), jnp.float32),
            pltpu.VMEM((PP, C), jnp.float32),
            pltpu.VMEM((K, D), jnp.float32),
            pltpu.VMEM((K, D), jnp.float32),
            pltpu.VMEM((HALF,), jnp.int32),
            pltpu.SemaphoreType.DMA,
            pltpu.SemaphoreType.DMA,
            pltpu.SemaphoreType.DMA,
            pltpu.SemaphoreType.DMA,
            pltpu.SemaphoreType.DMA,
            pltpu.SemaphoreType.DMA,
        ],
        compiler_params=pltpu.CompilerParams(needs_layout_passes=False),
    )()
    x_rows, tok_raw = sc_call(probs_t, embedding_weight)
    tok = tok_raw + jnp.asarray(tokens_shift, jnp.int32)
    x = jnp.transpose(x_rows.reshape(B, H, W, D), (0, 3, 1, 2))
    return (x, tok.reshape(B, H, W))


# speedup vs baseline: 1.9189x; 1.0793x over previous
import functools

import jax
import jax.numpy as jnp
from jax import lax
from jax.experimental import pallas as pl
from jax.experimental.pallas import tpu as pltpu
from jax.experimental.pallas import tpu_sc as plsc

B, C, H, W = 16, 1024, 32, 32
P = H * W
D = 256
L = 16
NC, NS = 2, 16
HALF = P // 2
PP = 16
NPC = HALF // PP
NV = C // L
K = 128
NK = HALF // K


def _sc_body(probs_hbm, emb_hbm, x_hbm, tok_hbm,
             pbuf0, pbuf1, rows0, rows1, bi,
             psem0, psem1, gsem0, gsem1, wsem0, wsem1):
    c = lax.axis_index("c")
    s = lax.axis_index("s")
    b = c * (B // NC) + s // 2
    p0 = (s % 2) * HALF

    iota = lax.iota(jnp.int32, L)

    def start_chunk(buf, sem, chunk_id):
        off = pl.multiple_of(p0 + chunk_id * PP, PP)
        pltpu.make_async_copy(
            probs_hbm.at[b, pl.ds(off, PP), :], buf, sem).start()

    def wait_chunk(buf, sem):
        pltpu.make_async_copy(
            probs_hbm.at[0, pl.ds(0, PP), :], buf, sem).wait()

    def process(buf, pbase):
        def pos_body(j):
            curv = buf[j, pl.ds(0, L)]
            curi = jnp.zeros((L,), jnp.int32)
            for v in range(1, NV):
                a = buf[j, pl.ds(v * L, L)]
                m = jnp.maximum(a, curv)
                upd = m != curv
                curi = jnp.where(upd, jnp.full((L,), v, jnp.int32), curi)
                curv = m
            gm = jnp.max(curv)
            chan = curi * L + iota
            cand = jnp.where(curv == jnp.full((L,), 1.0, jnp.float32) * gm,
                             chan, jnp.full((L,), C, jnp.int32))
            mn = jnp.full((L,), 1, jnp.int32) * jnp.min(cand)
            dst = jnp.full((L,), 1, jnp.int32) * (pbase + j)
            plsc.store_scatter(bi, [dst], mn, mask=iota == 0)

        plsc.parallel_loop(0, PP, unroll=4)(pos_body)

    start_chunk(pbuf0, psem0, 0)
    start_chunk(pbuf1, psem1, 1)

    def ring(g, _):
        wait_chunk(pbuf0, psem0)
        process(pbuf0, 2 * g * PP)

        @pl.when(2 * g + 2 < NPC)
        def _():
            start_chunk(pbuf0, psem0, 2 * g + 2)

        wait_chunk(pbuf1, psem1)
        process(pbuf1, (2 * g + 1) * PP)

        @pl.when(2 * g + 3 < NPC)
        def _():
            start_chunk(pbuf1, psem1, 2 * g + 3)

        return 0

    lax.fori_loop(0, NPC // 2, ring, 0)

    pltpu.sync_copy(bi, tok_hbm.at[b, pl.ds(p0, HALF)])

    rows = (rows0, rows1)
    gsems = (gsem0, gsem1)
    wsems = (wsem0, wsem1)
    wcopies = [None, None]
    for k in range(NK):
        par = k % 2
        if wcopies[par] is not None:
            wcopies[par].wait()
        pltpu.async_copy(
            emb_hbm.at[bi.at[pl.ds(k * K, K)]], rows[par], gsems[par]
        ).wait()
        wcopies[par] = pltpu.async_copy(
            rows[par], x_hbm.at[b, pl.ds(p0 + k * K, K), :], wsems[par])
    for wc in wcopies:
        if wc is not None:
            wc.wait()


def kernel(probs, tokens_shift, embedding_weight):
    probs_t = jnp.transpose(probs, (0, 2, 3, 1)).reshape(B, P, C)
    mesh = plsc.VectorSubcoreMesh(core_axis_name="c", subcore_axis_name="s")
    sc_call = functools.partial(
        pl.kernel, _sc_body, mesh=mesh,
        out_type=[
            jax.ShapeDtypeStruct((B, P, D), jnp.float32),
            jax.ShapeDtypeStruct((B, P), jnp.int32),
        ],
        scratch_types=[
            pltpu.VMEM((PP, C), jnp.float32),
            pltpu.VMEM((PP, C), jnp.float32),
            pltpu.VMEM((K, D), jnp.float32),
            pltpu.VMEM((K, D), jnp.float32),
            pltpu.VMEM((HALF,), jnp.int32),
            pltpu.SemaphoreType.DMA,
            pltpu.SemaphoreType.DMA,
            pltpu.SemaphoreType.DMA,
            pltpu.SemaphoreType.DMA,
            pltpu.SemaphoreType.DMA,
            pltpu.SemaphoreType.DMA,
        ],
        compiler_params=pltpu.CompilerParams(needs_layout_passes=False),
    )()
    x_rows, tok_raw = sc_call(probs_t, embedding_weight)
    tok = tok_raw + jnp.asarray(tokens_shift, jnp.int32)
    x = jnp.transpose(x_rows.reshape(B, H, W, D), (0, 3, 1, 2))
    return (x, tok.reshape(B, H, W))
